# baseline (speedup 1.0000x reference)
import jax
import jax.numpy as jnp
from jax import lax
from jax.experimental import pallas as pl
from jax.experimental.pallas import tpu as pltpu
from jax.experimental.pallas import tpu_sc as plsc

N = 10000
NNZ = 160000
F = 256
BLK = 64
NB = 2560
SUBCORES = 16
NCORES = 2
NW = NCORES * SUBCORES
RBPT = NB // SUBCORES
HALFR = 5120
RSRC = 10496
GSENT = 10240
AR = 5376
ART = AR // SUBCORES
OT = HALFR // SUBCORES
CW = 8
IC = 16
DEPTH = 2
CAPB = 164
CAPW = CAPB * BLK
ENTC = IC * BLK

_f32 = jnp.float32
_i32 = jnp.int32


def _prep_body(x_ref, wlin_ref, temb_ref, wtemb_ref, btemb_ref,
               xl_ref, tproj_ref):
    xl_ref[...] = jnp.zeros((RSRC, F), _f32)
    xl_ref[0:N, :] = jnp.dot(x_ref[...], wlin_ref[...].T,
                             preferred_element_type=_f32)
    t = temb_ref[...]
    st = t * (1.0 / (1.0 + jnp.exp(-t)))
    tproj_ref[...] = (
        jnp.dot(st, wtemb_ref[...].T, preferred_element_type=_f32)
        + btemb_ref[...])


_prep = pl.pallas_call(
    _prep_body,
    out_shape=(
        jax.ShapeDtypeStruct((RSRC, F), _f32),
        jax.ShapeDtypeStruct((1, F), _f32),
    ),
)


def _compact_body(gidx, sidx, out_g, out_s, out_len,
                  gstg, sstg, gcb, scb, lenb):
    c = lax.axis_index("c")
    s = lax.axis_index("s")
    w = c * SUBCORES + s
    lo = c * HALFR

    def _chunk(ch, off):
        base = s * RBPT + ch * 32
        pltpu.sync_copy(gidx.at[pl.ds(base, 32)], gstg)
        pltpu.sync_copy(sidx.at[pl.ds(base, 32)], sstg)

        def _row(r, off):
            for k in range(4):
                tv = sstg[r, pl.ds(k * 16, 16)]
                gv = gstg[r, pl.ds(k * 16, 16)]
                m = (tv >= lo) & (tv < lo + HALFR)
                nv = plsc.all_reduce_population_count(m)
                plsc.store_compressed(scb.at[pl.ds(off, 16)], tv - lo, mask=m)
                plsc.store_compressed(gcb.at[pl.ds(off, 16)], gv, mask=m)
                off = off + lax.reduce_max(nv, (0,))
            return off
        return lax.fori_loop(0, 32, _row, off)

    off = lax.fori_loop(0, RBPT // 32, _chunk, jnp.int32(0))

    nch = (off + ENTC - 1) // ENTC
    npadv = (nch * ENTC - off + 15) // 16
    fullm = jnp.ones((16,), jnp.bool_)

    def _pad(i, off):
        plsc.store_compressed(scb.at[pl.ds(off, 16)],
                              jnp.full((16,), HALFR, _i32), mask=fullm)
        plsc.store_compressed(gcb.at[pl.ds(off, 16)],
                              jnp.full((16,), GSENT, _i32), mask=fullm)
        return off + 16
    lax.fori_loop(0, npadv, _pad, off)

    lenb[...] = jnp.full((16,), nch, _i32)
    pltpu.sync_copy(gcb, out_g.at[w])
    pltpu.sync_copy(scb, out_s.at[w])
    pltpu.sync_copy(lenb, out_len.at[w])


_compact = pl.kernel(
    _compact_body,
    out_type=(
        jax.ShapeDtypeStruct((NW, CAPW), _i32),
        jax.ShapeDtypeStruct((NW, CAPW), _i32),
        jax.ShapeDtypeStruct((NW, 16), _i32),
    ),
    mesh=plsc.VectorSubcoreMesh(core_axis_name="c", subcore_axis_name="s"),
    scratch_types=[
        pltpu.VMEM((32, BLK), _i32),
        pltpu.VMEM((32, BLK), _i32),
        pltpu.VMEM((CAPW,), _i32),
        pltpu.VMEM((CAPW,), _i32),
        pltpu.VMEM((16,), _i32),
    ],
    compiler_params=pltpu.CompilerParams(use_tc_tiling_on_sc=False, needs_layout_passes=False),
)


def _accum_body(src, cg3, cs3, lens, ones_h, zr_h, zc_h,
                out, cnt_out,
                gstg, sstg, r0, r1, onesb, zbuf, zcnt, lenv, acc, cnt,
                s0, s1):
    rows = (r0, r1)
    sems = (s0, s1)
    c = lax.axis_index("c")
    s = lax.axis_index("s")
    w = c * SUBCORES + s
    rowlo = s * ART

    pltpu.sync_copy(zr_h, zbuf)
    pltpu.sync_copy(zc_h, zcnt)
    pltpu.sync_copy(ones_h, onesb)
    pltpu.sync_copy(lens.at[w], lenv)

    def _za(k, carry):
        pltpu.sync_copy(zbuf, acc.at[pl.ds(rowlo + k * 16, 16)])
        return carry
    lax.fori_loop(0, ART // 16, _za, 0)

    def _zc(k, carry):
        pltpu.sync_copy(zcnt, cnt.at[pl.ds(rowlo + k * 56, 56)])
        return carry
    lax.fori_loop(0, ART // 56, _zc, 0)
    plsc.subcore_barrier()

    nch = lax.reduce_max(lenv[...], (0,))

    def _chunk(ch, carry):
        pltpu.sync_copy(cg3.at[w, pl.ds(ch * IC, IC)], gstg)
        pltpu.sync_copy(cs3.at[w, pl.ds(ch * IC, IC)], sstg)
        for j in range(DEPTH):
            pltpu.async_copy(src.at[gstg.at[j]], rows[j], sems[j])

        def _grp(g, c2):
            for j in range(DEPTH):
                b = DEPTH * g + j
                pltpu.make_async_copy(
                    src.at[gstg.at[b]], rows[j], sems[j]).wait()
                pltpu.sync_copy(rows[j], acc.at[sstg.at[b]], add=True)
                pltpu.sync_copy(onesb, cnt.at[sstg.at[b]], add=True)

                @pl.when(b + DEPTH < IC)
                def _():
                    pltpu.async_copy(
                        src.at[gstg.at[b + DEPTH]], rows[j], sems[j])
            return c2
        lax.fori_loop(0, IC // DEPTH, _grp, 0)
        return carry
    lax.fori_loop(0, nch, _chunk, 0)

    plsc.subcore_barrier()
    pltpu.sync_copy(acc.at[pl.ds(s * OT, OT)],
                    out.at[pl.ds(c * HALFR + s * OT, OT)])
    pltpu.sync_copy(cnt.at[pl.ds(s * OT, OT)],
                    cnt_out.at[pl.ds(c * HALFR + s * OT, OT)])


_accum = pl.kernel(
    _accum_body,
    out_type=(
        jax.ShapeDtypeStruct((2 * HALFR, F), _f32),
        jax.ShapeDtypeStruct((2 * HALFR, CW), _f32),
    ),
    mesh=plsc.VectorSubcoreMesh(core_axis_name="c", subcore_axis_name="s"),
    scratch_types=[
        pltpu.VMEM((IC, BLK), _i32),
        pltpu.VMEM((IC, BLK), _i32),
        pltpu.VMEM((BLK, F), _f32),
        pltpu.VMEM((BLK, F), _f32),
        pltpu.VMEM((BLK, CW), _f32),
        pltpu.VMEM((16, F), _f32),
        pltpu.VMEM((56, CW), _f32),
        pltpu.VMEM((16,), _i32),
        pltpu.VMEM_SHARED((AR, F), _f32),
        pltpu.VMEM_SHARED((AR, CW), _f32),
        pltpu.SemaphoreType.DMA,
        pltpu.SemaphoreType.DMA,
    ],
    compiler_params=pltpu.CompilerParams(use_tc_tiling_on_sc=False, needs_layout_passes=False),
)


def _scale_body(a_ref, cnt_ref, e_ref):
    cnt = cnt_ref[...][:, 0:1]
    inv = jnp.where(cnt == 0, 0.0, 1.0 / cnt)
    e_ref[...] = jnp.zeros((RSRC, F), _f32)
    e_ref[0:2 * HALFR, :] = a_ref[...] * inv


_scale = pl.pallas_call(
    _scale_body,
    out_shape=jax.ShapeDtypeStruct((RSRC, F), _f32),
)


def _final_body(b_ref, cnt_ref, tproj_ref, out_ref):
    cnt = cnt_ref[0:N, 0:1]
    inv = jnp.where(cnt == 0, 0.0, 1.0 / cnt)
    h = b_ref[0:N, :] * inv + tproj_ref[...]
    out_ref[...] = h * (1.0 / (1.0 + jnp.exp(-h)))


_final = pl.pallas_call(
    _final_body,
    out_shape=jax.ShapeDtypeStruct((N, F), _f32),
)


def kernel(x, hyperedge_index, temb, W_lin, W_temb, b_temb):
    pad = jnp.full((NB * BLK - NNZ,), GSENT, _i32)
    nidx = jnp.concatenate([hyperedge_index[0], pad]).reshape(NB, BLK)
    eidx = jnp.concatenate([hyperedge_index[1], pad]).reshape(NB, BLK)
    ones_h = jnp.ones((BLK, CW), _f32)
    zr = jnp.zeros((16, F), _f32)
    zc = jnp.zeros((56, CW), _f32)

    xl, tproj = _prep(x, W_lin, temb, W_temb, b_temb.reshape(1, F))

    cg1, cs1, len1 = _compact(nidx, eidx)
    a, cntB = _accum(xl, cg1.reshape(NW, CAPB, BLK),
                     cs1.reshape(NW, CAPB, BLK), len1, ones_h, zr, zc)
    ef = _scale(a, cntB)

    cg2, cs2, len2 = _compact(eidx, nidx)
    b, cntD = _accum(ef, cg2.reshape(NW, CAPB, BLK),
                     cs2.reshape(NW, CAPB, BLK), len2, ones_h, zr, zc)
    return _final(b, cntD, tproj)


# speedup vs baseline: 3.5951x; 3.5951x over previous
import jax
import jax.numpy as jnp
from jax import lax
from jax.experimental import pallas as pl
from jax.experimental.pallas import tpu as pltpu
from jax.experimental.pallas import tpu_sc as plsc

N = 10000
NNZ = 160000
F = 256
HALF = 128
R = 10240
BLK = 64
NB = 2560
NNZ_PAD = NB * BLK - NNZ
SUBCORES = 16
BPT = NB // SUBCORES
IC = 32
NCH = BPT // IC
DEPTH = 4
RPT = R // SUBCORES
ZB = 32
ZC = 64
CW = 8

_f32 = jnp.float32
_bf16 = jnp.bfloat16


def _prep_body(x_ref, wlin_ref, temb_ref, wtemb_ref, btemb_ref,
               xl0_ref, xl1_ref, tproj_ref):
    xl = jnp.dot(x_ref[...], wlin_ref[...].T, preferred_element_type=_f32)
    xl0_ref[...] = jnp.zeros((R, HALF), _bf16)
    xl1_ref[...] = jnp.zeros((R, HALF), _bf16)
    xl0_ref[0:N, :] = xl[:, 0:HALF].astype(_bf16)
    xl1_ref[0:N, :] = xl[:, HALF:F].astype(_bf16)
    t = temb_ref[...]
    st = t * (1.0 / (1.0 + jnp.exp(-t)))
    tproj_ref[...] = (
        jnp.dot(st, wtemb_ref[...].T, preferred_element_type=_f32)
        + btemb_ref[...])


_prep = pl.pallas_call(
    _prep_body,
    out_shape=(
        jax.ShapeDtypeStruct((R, HALF), _bf16),
        jax.ShapeDtypeStruct((R, HALF), _bf16),
        jax.ShapeDtypeStruct((1, F), _f32),
    ),
)


def _make_sc_pass(with_counts):
    def body(src0, src1, gidx, sidx, ones_h, z128_h, z8_h,
             out0, out1, cs_out, cg_out,
             gloc, sloc, r0, r1, r2, r3, onesb, zbuf, zcnt, acc, cnt,
             s0, s1, s2, s3):
        rows = (r0, r1, r2, r3)
        sems = (s0, s1, s2, s3)
        c = lax.axis_index("c")
        s = lax.axis_index("s")
        rowlo = s * RPT

        pltpu.sync_copy(z128_h, zbuf)

        def _za(k, carry):
            pltpu.sync_copy(zbuf, acc.at[pl.ds(rowlo + k * ZB, ZB)])
            return carry
        lax.fori_loop(0, RPT // ZB, _za, 0)

        if with_counts:
            pltpu.sync_copy(z8_h, zcnt)
            pltpu.sync_copy(ones_h, onesb)

            def _zc(k, carry):
                pltpu.sync_copy(zcnt, cnt.at[pl.ds(rowlo + k * ZC, ZC)])
                return carry
            lax.fori_loop(0, RPT // ZC, _zc, 0)
        plsc.subcore_barrier()

        def _run(src, cidx):
            def _chunk(ch, carry):
                base = s * BPT + ch * IC
                pltpu.sync_copy(gidx.at[pl.ds(base, IC)], gloc)
                pltpu.sync_copy(sidx.at[pl.ds(base, IC)], sloc)
                for j in range(DEPTH):
                    pltpu.async_copy(src.at[gloc.at[j]], rows[j], sems[j])

                def _grp(g, c2):
                    for j in range(DEPTH):
                        b = DEPTH * g + j
                        pltpu.make_async_copy(
                            src.at[gloc.at[b]], rows[j], sems[j]).wait()
                        pltpu.sync_copy(
                            rows[j], acc.at[sloc.at[b]], add=True)
                        if with_counts:
                            pltpu.sync_copy(
                                onesb, cnt.at[cidx.at[b]], add=True)

                        @pl.when(b + DEPTH < IC)
                        def _():
                            pltpu.async_copy(
                                src.at[gloc.at[b + DEPTH]], rows[j], sems[j])
                    return c2
                lax.fori_loop(0, IC // DEPTH, _grp, 0)
                return carry
            lax.fori_loop(0, NCH, _chunk, 0)

        @pl.when(c == 0)
        def _():
            _run(src0, sloc)

        @pl.when(c == 1)
        def _():
            _run(src1, gloc)

        plsc.subcore_barrier()
        sl = pl.ds(rowlo, RPT)

        @pl.when(c == 0)
        def _():
            pltpu.sync_copy(acc.at[sl], out0.at[sl])
            if with_counts:
                pltpu.sync_copy(cnt.at[sl], cs_out.at[sl])

        @pl.when(c == 1)
        def _():
            pltpu.sync_copy(acc.at[sl], out1.at[sl])
            if with_counts:
                pltpu.sync_copy(cnt.at[sl], cg_out.at[sl])

    return pl.kernel(
        body,
        out_type=(
            jax.ShapeDtypeStruct((R, HALF), _bf16),
            jax.ShapeDtypeStruct((R, HALF), _bf16),
            jax.ShapeDtypeStruct((R, CW), _f32),
            jax.ShapeDtypeStruct((R, CW), _f32),
        ),
        mesh=plsc.VectorSubcoreMesh(core_axis_name="c", subcore_axis_name="s"),
        scratch_types=[
            pltpu.VMEM((IC, BLK), jnp.int32),
            pltpu.VMEM((IC, BLK), jnp.int32),
            pltpu.VMEM((BLK, HALF), _bf16),
            pltpu.VMEM((BLK, HALF), _bf16),
            pltpu.VMEM((BLK, HALF), _bf16),
            pltpu.VMEM((BLK, HALF), _bf16),
            pltpu.VMEM((BLK, CW), _f32),
            pltpu.VMEM((ZB, HALF), _bf16),
            pltpu.VMEM((ZC, CW), _f32),
            pltpu.VMEM_SHARED((R, HALF), _bf16),
            pltpu.VMEM_SHARED((R, CW), _f32),
            pltpu.SemaphoreType.DMA,
            pltpu.SemaphoreType.DMA,
            pltpu.SemaphoreType.DMA,
            pltpu.SemaphoreType.DMA,
        ],
        compiler_params=pltpu.CompilerParams(use_tc_tiling_on_sc=False),
    )


_sc_pass1 = _make_sc_pass(with_counts=True)
_sc_pass2 = _make_sc_pass(with_counts=False)


def _scale_body(a0_ref, a1_ref, cnt_ref, e0_ref, e1_ref):
    cnt = cnt_ref[...][:, 0:1]
    inv = jnp.where(cnt == 0, 0.0, 1.0 / cnt)
    e0_ref[...] = (a0_ref[...].astype(_f32) * inv).astype(_bf16)
    e1_ref[...] = (a1_ref[...].astype(_f32) * inv).astype(_bf16)


_scale = pl.pallas_call(
    _scale_body,
    out_shape=(
        jax.ShapeDtypeStruct((R, HALF), _bf16),
        jax.ShapeDtypeStruct((R, HALF), _bf16),
    ),
)


def _final_body(b0_ref, b1_ref, cnt_ref, tproj_ref, out_ref):
    cnt = cnt_ref[0:N, 0:1]
    inv = jnp.where(cnt == 0, 0.0, 1.0 / cnt)
    node_out = jnp.concatenate(
        [b0_ref[0:N, :].astype(_f32) * inv,
         b1_ref[0:N, :].astype(_f32) * inv], axis=1)
    h = node_out + tproj_ref[...]
    out_ref[...] = h * (1.0 / (1.0 + jnp.exp(-h)))


_final = pl.pallas_call(
    _final_body,
    out_shape=jax.ShapeDtypeStruct((N, F), _f32),
)


def kernel(x, hyperedge_index, temb, W_lin, W_temb, b_temb):
    pad = jnp.full((NNZ_PAD,), N, jnp.int32)
    nidx = jnp.concatenate([hyperedge_index[0], pad]).reshape(NB, BLK)
    eidx = jnp.concatenate([hyperedge_index[1], pad]).reshape(NB, BLK)
    ones_h = jnp.ones((BLK, CW), _f32)
    z128 = jnp.zeros((ZB, HALF), _bf16)
    z8 = jnp.zeros((ZC, CW), _f32)

    xl0, xl1, tproj = _prep(x, W_lin, temb, W_temb, b_temb.reshape(1, F))
    a0, a1, cntB, cntD = _sc_pass1(xl0, xl1, nidx, eidx, ones_h, z128, z8)
    ef0, ef1 = _scale(a0, a1, cntB)
    b0, b1, _, _ = _sc_pass2(ef0, ef1, eidx, nidx, ones_h, z128, z8)
    return _final(b0, b1, cntD, tproj)


# speedup vs baseline: 4.3376x; 1.2065x over previous
import jax
import jax.numpy as jnp
from jax import lax
from jax.experimental import pallas as pl
from jax.experimental.pallas import tpu as pltpu
from jax.experimental.pallas import tpu_sc as plsc

N = 10000
NNZ = 160000
F = 256
HALF = 128
R = 10240
BLK = 64
NB = 2560
NNZ_PAD = NB * BLK - NNZ
SUBCORES = 16
BPT = NB // SUBCORES
IC = 32
NCH = BPT // IC
DEPTH = 2
RPT = R // SUBCORES
ZB = 16
ZC = 64
CW = 16
SC_ROWS = 128

_f32 = jnp.float32
_bf16 = jnp.bfloat16
_i32 = jnp.int32


def _prep_body(x_ref, wlin_ref, temb_ref, wtemb_ref, btemb_ref,
               xl0_ref, xl1_ref, tproj_ref):
    xl = jnp.dot(x_ref[...], wlin_ref[...].T, preferred_element_type=_f32)
    xl0_ref[...] = jnp.zeros((R, HALF), _bf16)
    xl1_ref[...] = jnp.zeros((R, HALF), _bf16)
    xl0_ref[0:N, :] = xl[:, 0:HALF].astype(_bf16)
    xl1_ref[0:N, :] = xl[:, HALF:F].astype(_bf16)
    t = temb_ref[...]
    st = t * (1.0 / (1.0 + jnp.exp(-t)))
    tproj_ref[...] = (
        jnp.dot(st, wtemb_ref[...].T, preferred_element_type=_f32)
        + btemb_ref[...])


_prep = pl.pallas_call(
    _prep_body,
    out_shape=(
        jax.ShapeDtypeStruct((R, HALF), _bf16),
        jax.ShapeDtypeStruct((R, HALF), _bf16),
        jax.ShapeDtypeStruct((1, F), _f32),
    ),
)


def _sc_body(src0, src1, nidx, eidx, ones_h, zr_h, zc_h,
             out0, out1, cd_out,
             gloc, sloc, r0, r1, onesb, zbuf, zcnt, astg, cstg,
             acc1, acc2, cntB, cntD, s0, s1):
    rows = (r0, r1)
    sems = (s0, s1)
    c = lax.axis_index("c")
    s = lax.axis_index("s")
    rowlo = s * RPT

    pltpu.sync_copy(zr_h, zbuf)
    pltpu.sync_copy(zc_h, zcnt)
    pltpu.sync_copy(ones_h, onesb)

    def _za(k, carry):
        pltpu.sync_copy(zbuf, acc1.at[pl.ds(rowlo + k * ZB, ZB)])
        pltpu.sync_copy(zbuf, acc2.at[pl.ds(rowlo + k * ZB, ZB)])
        return carry
    lax.fori_loop(0, RPT // ZB, _za, 0)

    def _zc(k, carry):
        pltpu.sync_copy(zcnt, cntB.at[pl.ds(rowlo + k * ZC, ZC)])
        pltpu.sync_copy(zcnt, cntD.at[pl.ds(rowlo + k * ZC, ZC)])
        return carry
    lax.fori_loop(0, RPT // ZC, _zc, 0)
    plsc.subcore_barrier()

    def _loop(src, gidx, sidx, acc, cnt):
        def _chunk(ch, carry):
            base = s * BPT + ch * IC
            pltpu.sync_copy(gidx.at[pl.ds(base, IC)], gloc)
            pltpu.sync_copy(sidx.at[pl.ds(base, IC)], sloc)
            for j in range(DEPTH):
                pltpu.async_copy(src.at[gloc.at[j]], rows[j], sems[j])

            def _grp(g, c2):
                for j in range(DEPTH):
                    b = DEPTH * g + j
                    pltpu.make_async_copy(
                        src.at[gloc.at[b]], rows[j], sems[j]).wait()
                    pltpu.sync_copy(rows[j], acc.at[sloc.at[b]], add=True)
                    pltpu.sync_copy(onesb, cnt.at[sloc.at[b]], add=True)

                    @pl.when(b + DEPTH < IC)
                    def _():
                        pltpu.async_copy(
                            src.at[gloc.at[b + DEPTH]], rows[j], sems[j])
                return c2
            lax.fori_loop(0, IC // DEPTH, _grp, 0)
            return carry
        lax.fori_loop(0, NCH, _chunk, 0)

    @pl.when(c == 0)
    def _():
        _loop(src0, nidx, eidx, acc1, cntB)

    @pl.when(c == 1)
    def _():
        _loop(src1, nidx, eidx, acc1, cntB)

    plsc.subcore_barrier()

    def _sch(q, carry):
        base = rowlo + q * SC_ROWS
        pltpu.sync_copy(acc1.at[pl.ds(base, SC_ROWS)], astg)
        pltpu.sync_copy(cntB.at[pl.ds(base, SC_ROWS)], cstg)

        def _srow(r, c2):
            cv = cstg[r, pl.ds(0, 16)]
            invv = jnp.where(cv == 0.0, 0.0, 1.0 / cv)
            sv = jnp.full((16,), lax.reduce_max(invv, (0,)), _f32)
            sb = plsc.pack(sv, sv, format=plsc.PackFormat.INTERLEAVED)
            for k in range(HALF // 32):
                sl = pl.ds(k * 32, 32)
                astg[r, sl] = astg[r, sl] * sb
            return c2
        lax.fori_loop(0, SC_ROWS, _srow, 0)
        pltpu.sync_copy(astg, acc1.at[pl.ds(base, SC_ROWS)])
        return carry
    lax.fori_loop(0, RPT // SC_ROWS, _sch, 0)
    plsc.subcore_barrier()

    _loop(acc1, eidx, nidx, acc2, cntD)
    plsc.subcore_barrier()

    sl = pl.ds(rowlo, RPT)

    @pl.when(c == 0)
    def _():
        pltpu.sync_copy(acc2.at[sl], out0.at[sl])
        pltpu.sync_copy(cntD.at[sl], cd_out.at[sl])

    @pl.when(c == 1)
    def _():
        pltpu.sync_copy(acc2.at[sl], out1.at[sl])


_sc_fused = pl.kernel(
    _sc_body,
    out_type=(
        jax.ShapeDtypeStruct((R, HALF), _bf16),
        jax.ShapeDtypeStruct((R, HALF), _bf16),
        jax.ShapeDtypeStruct((R, CW), _f32),
    ),
    mesh=plsc.VectorSubcoreMesh(core_axis_name="c", subcore_axis_name="s"),
    scratch_types=[
        pltpu.VMEM((IC, BLK), _i32),
        pltpu.VMEM((IC, BLK), _i32),
        pltpu.VMEM((BLK, HALF), _bf16),
        pltpu.VMEM((BLK, HALF), _bf16),
        pltpu.VMEM((BLK, CW), _f32),
        pltpu.VMEM((ZB, HALF), _bf16),
        pltpu.VMEM((ZC, CW), _f32),
        pltpu.VMEM((SC_ROWS, HALF), _bf16),
        pltpu.VMEM((SC_ROWS, CW), _f32),
        pltpu.VMEM_SHARED((R, HALF), _bf16),
        pltpu.VMEM_SHARED((R, HALF), _bf16),
        pltpu.VMEM_SHARED((R, CW), _f32),
        pltpu.VMEM_SHARED((R, CW), _f32),
        pltpu.SemaphoreType.DMA,
        pltpu.SemaphoreType.DMA,
    ],
    compiler_params=pltpu.CompilerParams(
        use_tc_tiling_on_sc=False, needs_layout_passes=False),
)


def _final_body(b0_ref, b1_ref, cnt_ref, tproj_ref, out_ref):
    cnt = cnt_ref[0:N, 0:1]
    inv = jnp.where(cnt == 0, 0.0, 1.0 / cnt)
    node_out = jnp.concatenate(
        [b0_ref[0:N, :].astype(_f32) * inv,
         b1_ref[0:N, :].astype(_f32) * inv], axis=1)
    h = node_out + tproj_ref[...]
    out_ref[...] = h * (1.0 / (1.0 + jnp.exp(-h)))


_final = pl.pallas_call(
    _final_body,
    out_shape=jax.ShapeDtypeStruct((N, F), _f32),
)


def kernel(x, hyperedge_index, temb, W_lin, W_temb, b_temb):
    pad = jnp.full((NNZ_PAD,), N, _i32)
    nidx = jnp.concatenate([hyperedge_index[0], pad]).reshape(NB, BLK)
    eidx = jnp.concatenate([hyperedge_index[1], pad]).reshape(NB, BLK)
    ones_h = jnp.ones((BLK, CW), _f32)
    zr = jnp.zeros((ZB, HALF), _bf16)
    zc = jnp.zeros((ZC, CW), _f32)

    xl0, xl1, tproj = _prep(x, W_lin, temb, W_temb, b_temb.reshape(1, F))
    b0, b1, cntD = _sc_fused(xl0, xl1, nidx, eidx, ones_h, zr, zc)
    return _final(b0, b1, cntD, tproj)
